# baseline (speedup 1.0000x reference)
import jax
import jax.numpy as jnp
from jax.experimental import pallas as pl

MU_ = 256
ROWS_PER_BLOCK = 8


def _onehot_body(x_ref, o_ref):
    idx = ((x_ref[...] + 1.0) * 128.0).astype(jnp.int32)
    iota = jax.lax.broadcasted_iota(jnp.int32, (ROWS_PER_BLOCK, MU_, MU_), 2)
    o_ref[...] = (idx[:, :, None] == iota).astype(jnp.float32)


def kernel(x):
    b, t, _ = x.shape
    n = b * t
    xr = x.reshape(n // MU_, MU_)
    grid = (n // MU_ // ROWS_PER_BLOCK,)
    out = pl.pallas_call(
        _onehot_body,
        grid=grid,
        in_specs=[pl.BlockSpec((ROWS_PER_BLOCK, MU_), lambda i: (i, 0))],
        out_specs=pl.BlockSpec((ROWS_PER_BLOCK, MU_, MU_), lambda i: (i, 0, 0)),
        out_shape=jax.ShapeDtypeStruct((n // MU_, MU_, MU_), jnp.float32),
    )(xr)
    return out.reshape(b, t, MU_)
